# Initial kernel scaffold; baseline (speedup 1.0000x reference)
import jax
import jax.numpy as jnp
from jax.experimental import pallas as pl


def kernel(h, edge_index):
    raise NotImplementedError("write your pallas kernel here")


# speedup vs baseline: 1.8959x; 1.8959x over previous
import functools

import jax
import jax.numpy as jnp
from jax import lax
from jax.experimental import pallas as pl
from jax.experimental.pallas import tpu as pltpu
from jax.experimental.pallas import tpu_sc as plsc

NC, NS = 2, 16
NW = NC * NS


def _gather_body(h_hbm, idx_hbm, out_hbm, idx_v, rows_v, sem, *, nch, ch):
    wid = lax.axis_index("s") * NC + lax.axis_index("c")
    pltpu.sync_copy(idx_hbm.at[wid], idx_v)

    def step(g, carry):
        pltpu.async_copy(h_hbm.at[idx_v.at[g]], rows_v, sem).wait()
        pltpu.sync_copy(rows_v, out_hbm.at[pl.ds((wid * nch + g) * ch, ch)])
        return carry

    lax.fori_loop(0, nch, step, 0)


def kernel(h, edge_index):
    n, d = h.shape
    e = edge_index.shape[1]
    r = 2 * e
    per_w = r // NW
    ch = 80
    nch = per_w // ch

    idx = edge_index.astype(jnp.int32).T.reshape(NW, nch, ch)

    body = functools.partial(_gather_body, nch=nch, ch=ch)
    out = pl.kernel(
        body,
        out_type=jax.ShapeDtypeStruct((r, d), jnp.float32),
        mesh=plsc.VectorSubcoreMesh(core_axis_name="c", subcore_axis_name="s"),
        scratch_types=[
            pltpu.VMEM((nch, ch), jnp.int32),
            pltpu.VMEM((ch, d), jnp.float32),
            pltpu.SemaphoreType.DMA,
        ],
    )(h, idx)
    return out.reshape(e, 2 * d)


# speedup vs baseline: 2.0432x; 1.0777x over previous
import functools

import jax
import jax.numpy as jnp
from jax import lax
from jax.experimental import pallas as pl
from jax.experimental.pallas import tpu as pltpu
from jax.experimental.pallas import tpu_sc as plsc

NC, NS = 2, 16
NW = NC * NS


def _gather_body(h_hbm, idx_hbm, out_hbm, idx_v, rows0, rows1, sg0, sg1,
                 sw0, sw1, *, nch, ch):
    wid = lax.axis_index("s") * NC + lax.axis_index("c")
    pltpu.sync_copy(idx_hbm.at[wid], idx_v)
    base = wid * nch

    def gather(g, rows, sem):
        return pltpu.make_async_copy(h_hbm.at[idx_v.at[g]], rows, sem)

    def writeback(g, rows, sem):
        return pltpu.make_async_copy(rows, out_hbm.at[pl.ds((base + g) * ch, ch)], sem)

    def slot(g, rows_b, sem_gb, sem_wb, rows_o, sem_go, sem_wo):
        gather(g, rows_b, sem_gb).wait()
        writeback(g, rows_b, sem_wb).start()

        @pl.when(g > 0)
        def _():
            writeback(g - 1, rows_o, sem_wo).wait()

        @pl.when(g + 1 < nch)
        def _():
            gather(g + 1, rows_o, sem_go).start()

    def pair(t, carry):
        g0 = 2 * t
        slot(g0, rows0, sg0, sw0, rows1, sg1, sw1)
        slot(g0 + 1, rows1, sg1, sw1, rows0, sg0, sw0)
        return carry

    gather(0, rows0, sg0).start()
    lax.fori_loop(0, nch // 2, pair, 0)
    writeback(nch - 1, rows1, sw1).wait()


def kernel(h, edge_index):
    n, d = h.shape
    e = edge_index.shape[1]
    r = 2 * e
    per_w = r // NW
    ch = 80
    nch = per_w // ch

    idx = edge_index.astype(jnp.int32).T.reshape(NW, nch, ch)

    body = functools.partial(_gather_body, nch=nch, ch=ch)
    out = pl.kernel(
        body,
        out_type=jax.ShapeDtypeStruct((r, d), jnp.float32),
        mesh=plsc.VectorSubcoreMesh(core_axis_name="c", subcore_axis_name="s"),
        scratch_types=[
            pltpu.VMEM((nch, ch), jnp.int32),
            pltpu.VMEM((ch, d), jnp.float32),
            pltpu.VMEM((ch, d), jnp.float32),
            pltpu.SemaphoreType.DMA,
            pltpu.SemaphoreType.DMA,
            pltpu.SemaphoreType.DMA,
            pltpu.SemaphoreType.DMA,
        ],
    )(h, idx)
    return out.reshape(e, 2 * d)


# speedup vs baseline: 2.6771x; 1.3102x over previous
import functools

import jax
import jax.numpy as jnp
from jax import lax
from jax.experimental import pallas as pl
from jax.experimental.pallas import tpu as pltpu
from jax.experimental.pallas import tpu_sc as plsc

NC, NS = 2, 16
NW = NC * NS


def _gather_body(h_hbm, idx_hbm, out_hbm, h_sp, idx_v, rows0, rows1, sg0, sg1,
                 sw0, sw1, *, nch, ch, rows_per_tile):
    cid = lax.axis_index("c")
    sid = lax.axis_index("s")
    wid = sid * NC + cid
    pltpu.sync_copy(idx_hbm.at[wid], idx_v)
    n_rows = h_sp.shape[0]
    off = pl.multiple_of(jnp.minimum(sid * rows_per_tile, n_rows - rows_per_tile), 8)
    pltpu.sync_copy(h_hbm.at[pl.ds(off, rows_per_tile)],
                    h_sp.at[pl.ds(off, rows_per_tile)])
    plsc.subcore_barrier()
    base = wid * nch

    def gather(g, rows, sem):
        return pltpu.make_async_copy(h_sp.at[idx_v.at[pl.ds(g * ch, ch)]], rows, sem)

    def writeback(g, rows, sem):
        return pltpu.make_async_copy(rows, out_hbm.at[pl.ds((base + g) * ch, ch)], sem)

    def slot(g, rows_b, sem_gb, sem_wb, rows_o, sem_go, sem_wo):
        gather(g, rows_b, sem_gb).wait()
        writeback(g, rows_b, sem_wb).start()

        @pl.when(g > 0)
        def _():
            writeback(g - 1, rows_o, sem_wo).wait()

        @pl.when(g + 1 < nch)
        def _():
            gather(g + 1, rows_o, sem_go).start()

    def pair(t, carry):
        g0 = 2 * t
        slot(g0, rows0, sg0, sw0, rows1, sg1, sw1)
        slot(g0 + 1, rows1, sg1, sw1, rows0, sg0, sw0)
        return carry

    gather(0, rows0, sg0).start()
    lax.fori_loop(0, nch // 2, pair, 0)
    writeback(nch - 1, rows1, sw1).wait()


def kernel(h, edge_index):
    n, d = h.shape
    e = edge_index.shape[1]
    r = 2 * e
    per_w = r // NW
    ch = 80
    nch = per_w // ch

    idx = edge_index.astype(jnp.int32).T.reshape(NW, per_w)

    rpt = ((n + NS - 1) // NS + 7) // 8 * 8
    body = functools.partial(_gather_body, nch=nch, ch=ch, rows_per_tile=rpt)
    out = pl.kernel(
        body,
        out_type=jax.ShapeDtypeStruct((r, d), jnp.float32),
        mesh=plsc.VectorSubcoreMesh(core_axis_name="c", subcore_axis_name="s"),
        scratch_types=[
            pltpu.VMEM_SHARED((n, d), jnp.float32),
            pltpu.VMEM((per_w,), jnp.int32),
            pltpu.VMEM((ch, d), jnp.float32),
            pltpu.VMEM((ch, d), jnp.float32),
            pltpu.SemaphoreType.DMA,
            pltpu.SemaphoreType.DMA,
            pltpu.SemaphoreType.DMA,
            pltpu.SemaphoreType.DMA,
        ],
    )(h, idx)
    return out.reshape(e, 2 * d)


# speedup vs baseline: 11.2032x; 4.1849x over previous
import functools

import jax
import jax.numpy as jnp
from jax import lax
from jax.experimental import pallas as pl
from jax.experimental.pallas import tpu as pltpu
from jax.experimental.pallas import tpu_sc as plsc

NC, NS = 2, 16
NW = NC * NS


def _gather_body(h_hbm, ei_hbm, out_hbm, h_sp, src_v, dst_v,
                 bs0, bd0, bs1, bd1, sg0, sg1, sw0, sw1,
                 *, nch, ec, e_per_w, rows_per_tile):
    cid = lax.axis_index("c")
    sid = lax.axis_index("s")
    wid = sid * NC + cid
    e0 = wid * e_per_w
    d = h_sp.shape[1]
    n_edges = ei_hbm.shape[0] // 2
    n_rows = h_sp.shape[0]
    off = pl.multiple_of(jnp.minimum(sid * rows_per_tile, n_rows - rows_per_tile), 8)
    pltpu.sync_copy(h_hbm.at[pl.ds(off, rows_per_tile)],
                    h_sp.at[pl.ds(off, rows_per_tile)])
    pltpu.sync_copy(ei_hbm.at[pl.ds(e0, e_per_w)], src_v)
    pltpu.sync_copy(ei_hbm.at[pl.ds(n_edges + e0, e_per_w)], dst_v)
    plsc.subcore_barrier()

    def gathers(g, bs, bd, sem):
        sl = pl.ds(g * ec, ec)
        return (pltpu.make_async_copy(h_sp.at[src_v.at[sl]], bs, sem),
                pltpu.make_async_copy(h_sp.at[dst_v.at[sl]], bd, sem))

    def writebacks(g, bs, bd, sem):
        sl = pl.ds(e0 + g * ec, ec)
        return (pltpu.make_async_copy(bs, out_hbm.at[sl, pl.ds(0, d)], sem),
                pltpu.make_async_copy(bd, out_hbm.at[sl, pl.ds(d, d)], sem))

    def start(pair):
        pair[0].start()
        pair[1].start()

    def wait(pair):
        pair[0].wait()
        pair[1].wait()

    def slot(g, bs_b, bd_b, sem_gb, sem_wb, bs_o, bd_o, sem_go, sem_wo):
        wait(gathers(g, bs_b, bd_b, sem_gb))
        start(writebacks(g, bs_b, bd_b, sem_wb))

        @pl.when(g > 0)
        def _():
            wait(writebacks(g - 1, bs_o, bd_o, sem_wo))

        @pl.when(g + 1 < nch)
        def _():
            start(gathers(g + 1, bs_o, bd_o, sem_go))

    def pair(t, carry):
        g0 = 2 * t
        slot(g0, bs0, bd0, sg0, sw0, bs1, bd1, sg1, sw1)
        slot(g0 + 1, bs1, bd1, sg1, sw1, bs0, bd0, sg0, sw0)
        return carry

    start(gathers(0, bs0, bd0, sg0))
    lax.fori_loop(0, nch // 2, pair, 0)
    wait(writebacks(nch - 1, bs1, bd1, sw1))


def kernel(h, edge_index):
    n, d = h.shape
    e = edge_index.shape[1]
    e_per_w = e // NW
    ec = 40
    nch = e_per_w // ec

    ei = edge_index.astype(jnp.int32).reshape(-1)
    rpt = ((n + NS - 1) // NS + 7) // 8 * 8
    body = functools.partial(_gather_body, nch=nch, ec=ec, e_per_w=e_per_w,
                             rows_per_tile=rpt)
    return pl.kernel(
        body,
        out_type=jax.ShapeDtypeStruct((e, 2 * d), jnp.float32),
        mesh=plsc.VectorSubcoreMesh(core_axis_name="c", subcore_axis_name="s"),
        scratch_types=[
            pltpu.VMEM_SHARED((n, d), jnp.float32),
            pltpu.VMEM((e_per_w,), jnp.int32),
            pltpu.VMEM((e_per_w,), jnp.int32),
            pltpu.VMEM((ec, d), jnp.float32),
            pltpu.VMEM((ec, d), jnp.float32),
            pltpu.VMEM((ec, d), jnp.float32),
            pltpu.VMEM((ec, d), jnp.float32),
            pltpu.SemaphoreType.DMA,
            pltpu.SemaphoreType.DMA,
            pltpu.SemaphoreType.DMA,
            pltpu.SemaphoreType.DMA,
        ],
    )(h, ei)


# speedup vs baseline: 12.2348x; 1.0921x over previous
import functools

import jax
import jax.numpy as jnp
from jax import lax
from jax.experimental import pallas as pl
from jax.experimental.pallas import tpu as pltpu
from jax.experimental.pallas import tpu_sc as plsc

NC, NS = 2, 16
NW = NC * NS


def _gather_body(h_hbm, ei_hbm, out_hbm, h_sp, src_v, dst_v,
                 bs0, bd0, bs1, bd1, bs2, bd2, sg0, sg1, sg2, sw0, sw1, sw2,
                 *, nch, ec, e_per_w, rows_per_tile):
    cid = lax.axis_index("c")
    sid = lax.axis_index("s")
    wid = sid * NC + cid
    e0 = wid * e_per_w
    d = h_sp.shape[1]
    n_edges = ei_hbm.shape[0] // 2
    n_rows = h_sp.shape[0]
    off = pl.multiple_of(jnp.minimum(sid * rows_per_tile, n_rows - rows_per_tile), 8)
    pltpu.sync_copy(h_hbm.at[pl.ds(off, rows_per_tile)],
                    h_sp.at[pl.ds(off, rows_per_tile)])
    pltpu.sync_copy(ei_hbm.at[pl.ds(e0, e_per_w)], src_v)
    pltpu.sync_copy(ei_hbm.at[pl.ds(n_edges + e0, e_per_w)], dst_v)
    plsc.subcore_barrier()

    def gathers(g, bs, bd, sem):
        sl = pl.ds(g * ec, ec)
        return (pltpu.make_async_copy(h_sp.at[src_v.at[sl]], bs, sem),
                pltpu.make_async_copy(h_sp.at[dst_v.at[sl]], bd, sem))

    def writebacks(g, bs, bd, sem):
        sl = pl.ds(e0 + g * ec, ec)
        return (pltpu.make_async_copy(bs, out_hbm.at[sl, pl.ds(0, d)], sem),
                pltpu.make_async_copy(bd, out_hbm.at[sl, pl.ds(d, d)], sem))

    def start(pair):
        pair[0].start()
        pair[1].start()

    def wait(pair):
        pair[0].wait()
        pair[1].wait()

    bufs = ((bs0, bd0, sg0, sw0), (bs1, bd1, sg1, sw1), (bs2, bd2, sg2, sw2))

    def slot(g, cur, prev):
        cbs, cbd, csg, csw = cur
        pbs, pbd, psg, psw = prev
        wait(gathers(g, cbs, cbd, csg))
        start(writebacks(g, cbs, cbd, csw))

        @pl.when(g > 0)
        def _():
            wait(writebacks(g - 1, pbs, pbd, psw))

        @pl.when(g + 2 < nch)
        def _():
            start(gathers(g + 2, pbs, pbd, psg))

    def triple(t, carry):
        g0 = 3 * t
        slot(g0, bufs[0], bufs[2])
        slot(g0 + 1, bufs[1], bufs[0])
        slot(g0 + 2, bufs[2], bufs[1])
        return carry

    start(gathers(0, bs0, bd0, sg0))
    start(gathers(1, bs1, bd1, sg1))
    lax.fori_loop(0, nch // 3, triple, 0)
    glast = nch - 1
    wait(gathers(glast, bs0, bd0, sg0))
    start(writebacks(glast, bs0, bd0, sw0))
    wait(writebacks(glast - 1, bs2, bd2, sw2))
    wait(writebacks(glast, bs0, bd0, sw0))


def kernel(h, edge_index):
    n, d = h.shape
    e = edge_index.shape[1]
    e_per_w = e // NW
    ec = 40
    nch = e_per_w // ec

    ei = edge_index.astype(jnp.int32).reshape(-1)
    rpt = ((n + NS - 1) // NS + 7) // 8 * 8
    body = functools.partial(_gather_body, nch=nch, ec=ec, e_per_w=e_per_w,
                             rows_per_tile=rpt)
    return pl.kernel(
        body,
        out_type=jax.ShapeDtypeStruct((e, 2 * d), jnp.float32),
        mesh=plsc.VectorSubcoreMesh(core_axis_name="c", subcore_axis_name="s"),
        scratch_types=[
            pltpu.VMEM_SHARED((n, d), jnp.float32),
            pltpu.VMEM((e_per_w,), jnp.int32),
            pltpu.VMEM((e_per_w,), jnp.int32),
            pltpu.VMEM((ec, d), jnp.float32),
            pltpu.VMEM((ec, d), jnp.float32),
            pltpu.VMEM((ec, d), jnp.float32),
            pltpu.VMEM((ec, d), jnp.float32),
            pltpu.VMEM((ec, d), jnp.float32),
            pltpu.VMEM((ec, d), jnp.float32),
            pltpu.SemaphoreType.DMA,
            pltpu.SemaphoreType.DMA,
            pltpu.SemaphoreType.DMA,
            pltpu.SemaphoreType.DMA,
            pltpu.SemaphoreType.DMA,
            pltpu.SemaphoreType.DMA,
        ],
    )(h, ei)


# speedup vs baseline: 12.3878x; 1.0125x over previous
import functools

import jax
import jax.numpy as jnp
from jax import lax
from jax.experimental import pallas as pl
from jax.experimental.pallas import tpu as pltpu
from jax.experimental.pallas import tpu_sc as plsc

NC, NS = 2, 16
NW = NC * NS


def _gather_body(h_hbm, ei_hbm, out_hbm, h_sp, src_v, dst_v,
                 bs0, bd0, bs1, bd1, bs2, bd2, sg0, sg1, sg2, sw0, sw1, sw2,
                 *, nch, ec, e_per_w, rows_per_tile):
    cid = lax.axis_index("c")
    sid = lax.axis_index("s")
    wid = sid * NC + cid
    e0 = wid * e_per_w
    d = h_sp.shape[1]
    n_edges = ei_hbm.shape[0] // 2
    n_rows = h_sp.shape[0]
    off = pl.multiple_of(jnp.minimum(sid * rows_per_tile, n_rows - rows_per_tile), 8)
    cp_h = pltpu.make_async_copy(h_hbm.at[pl.ds(off, rows_per_tile)],
                                 h_sp.at[pl.ds(off, rows_per_tile)], sg0)
    cp_s = pltpu.make_async_copy(ei_hbm.at[pl.ds(e0, e_per_w)], src_v, sg1)
    cp_d = pltpu.make_async_copy(ei_hbm.at[pl.ds(n_edges + e0, e_per_w)], dst_v, sg2)
    cp_h.start()
    cp_s.start()
    cp_d.start()
    cp_h.wait()
    cp_s.wait()
    cp_d.wait()
    plsc.subcore_barrier()

    def gathers(g, bs, bd, sem):
        sl = pl.ds(g * ec, ec)
        return (pltpu.make_async_copy(h_sp.at[src_v.at[sl]], bs, sem),
                pltpu.make_async_copy(h_sp.at[dst_v.at[sl]], bd, sem))

    def writebacks(g, bs, bd, sem):
        sl = pl.ds(e0 + g * ec, ec)
        return (pltpu.make_async_copy(bs, out_hbm.at[sl, pl.ds(0, d)], sem),
                pltpu.make_async_copy(bd, out_hbm.at[sl, pl.ds(d, d)], sem))

    def start(pair):
        pair[0].start()
        pair[1].start()

    def wait(pair):
        pair[0].wait()
        pair[1].wait()

    bufs = ((bs0, bd0, sg0, sw0), (bs1, bd1, sg1, sw1), (bs2, bd2, sg2, sw2))

    def slot(g, cur, prev):
        cbs, cbd, csg, csw = cur
        pbs, pbd, psg, psw = prev
        wait(gathers(g, cbs, cbd, csg))
        start(writebacks(g, cbs, cbd, csw))

        @pl.when(g > 0)
        def _():
            wait(writebacks(g - 1, pbs, pbd, psw))

        @pl.when(g + 2 < nch)
        def _():
            start(gathers(g + 2, pbs, pbd, psg))

    def triple(t, carry):
        g0 = 3 * t
        slot(g0, bufs[0], bufs[2])
        slot(g0 + 1, bufs[1], bufs[0])
        slot(g0 + 2, bufs[2], bufs[1])
        return carry

    start(gathers(0, bs0, bd0, sg0))
    start(gathers(1, bs1, bd1, sg1))
    lax.fori_loop(0, nch // 3, triple, 0)
    glast = nch - 1
    wait(gathers(glast, bs0, bd0, sg0))
    start(writebacks(glast, bs0, bd0, sw0))
    wait(writebacks(glast - 1, bs2, bd2, sw2))
    wait(writebacks(glast, bs0, bd0, sw0))


def kernel(h, edge_index):
    n, d = h.shape
    e = edge_index.shape[1]
    e_per_w = e // NW
    ec = 40
    nch = e_per_w // ec

    ei = edge_index.astype(jnp.int32).reshape(-1)
    rpt = ((n + NS - 1) // NS + 7) // 8 * 8
    body = functools.partial(_gather_body, nch=nch, ec=ec, e_per_w=e_per_w,
                             rows_per_tile=rpt)
    return pl.kernel(
        body,
        out_type=jax.ShapeDtypeStruct((e, 2 * d), jnp.float32),
        mesh=plsc.VectorSubcoreMesh(core_axis_name="c", subcore_axis_name="s"),
        scratch_types=[
            pltpu.VMEM_SHARED((n, d), jnp.float32),
            pltpu.VMEM((e_per_w,), jnp.int32),
            pltpu.VMEM((e_per_w,), jnp.int32),
            pltpu.VMEM((ec, d), jnp.float32),
            pltpu.VMEM((ec, d), jnp.float32),
            pltpu.VMEM((ec, d), jnp.float32),
            pltpu.VMEM((ec, d), jnp.float32),
            pltpu.VMEM((ec, d), jnp.float32),
            pltpu.VMEM((ec, d), jnp.float32),
            pltpu.SemaphoreType.DMA,
            pltpu.SemaphoreType.DMA,
            pltpu.SemaphoreType.DMA,
            pltpu.SemaphoreType.DMA,
            pltpu.SemaphoreType.DMA,
            pltpu.SemaphoreType.DMA,
        ],
    )(h, ei)
